# baseline (speedup 1.0000x reference)
import functools

import jax
import jax.numpy as jnp
from jax import lax
from jax.experimental import pallas as pl
from jax.experimental.pallas import tpu as pltpu
from jax.experimental.pallas import tpu_sc as plsc

N_NODES = 10000
IN_DIM = 256
HIDDEN = 128
HWORDS = HIDDEN // 2
NUM_LAYERS = 2
NUM_PATHS = 4
PATH_LEN = 4
ALPHA = 0.1
BETA = 0.2

_INFO = plsc.get_sparse_core_info()
_NC = _INFO.num_cores
_NS = _INFO.num_subcores
_NW = _NC * _NS

N_PAD = 10240
FAN = NUM_PATHS * PATH_LEN
CHUNK_NODES = 8
ROWS_PER_CHUNK = CHUNK_NODES * FAN
TOT_CHUNKS = N_PAD // CHUNK_NODES
IDX_W = CHUNK_NODES * PATH_LEN
NBUF = 4
CH0 = 64
CH1 = 16
CHMAX = max(CH0, CH1)


@functools.partial(
    pl.kernel,
    mesh=plsc.VectorSubcoreMesh(core_axis_name="c", subcore_axis_name="s"),
    compiler_params=pltpu.CompilerParams(needs_layout_passes=False,
                                         use_tc_tiling_on_sc=False),
    out_type=jax.ShapeDtypeStruct((N_PAD, HIDDEN), jnp.float32),
    scratch_types=[
        pltpu.VMEM((CHMAX, ROWS_PER_CHUNK), jnp.int32),
        pltpu.VMEM((FAN, HIDDEN), jnp.float32),
        pltpu.VMEM((NBUF, ROWS_PER_CHUNK, HWORDS), jnp.int32),
        pltpu.VMEM((CHMAX * CHUNK_NODES, HIDDEN), jnp.float32),
    ] + [pltpu.SemaphoreType.DMA] * NBUF,
)
def _gather_reduce(pk_hbm, idx_hbm, coef_hbm, out_hbm,
                   idx_v, coef_v, rows_v, out_v, *sems):
    sid = lax.axis_index("s")
    cid = lax.axis_index("c")
    base = sid * (CH0 + CH1) + cid * CH0
    nchunks = jnp.where(cid == 0, CH0, CH1)

    @pl.when(cid == 0)
    def _():
        pltpu.sync_copy(idx_hbm.at[pl.ds(base, CH0)], idx_v.at[pl.ds(0, CH0)])

    @pl.when(cid == 1)
    def _():
        pltpu.sync_copy(idx_hbm.at[pl.ds(base, CH1)], idx_v.at[pl.ds(0, CH1)])

    pltpu.sync_copy(coef_hbm, coef_v)

    def unpack2(w):
        return plsc.unpack(plsc.bitcast(w, jnp.bfloat16),
                           format=plsc.PackFormat.INTERLEAVED)

    def compute(j, rows):
        for v in range(HWORDS // 16):
            sw = pl.ds(v * 16, 16)
            sl_lo = pl.ds(v * 16, 16)
            sl_hi = pl.ds(HWORDS + v * 16, 16)
            clo = [coef_v[r, sl_lo] for r in range(FAN)]
            chi = [coef_v[r, sl_hi] for r in range(FAN)]

            def node_body(c, carry):
                lo, hi = unpack2(rows[c * FAN, sw])
                acc_lo = clo[0] * lo
                acc_hi = chi[0] * hi
                for r in range(1, FAN):
                    lo, hi = unpack2(rows[c * FAN + r, sw])
                    acc_lo = acc_lo + clo[r] * lo
                    acc_hi = acc_hi + chi[r] * hi
                out_v[j * CHUNK_NODES + c, sl_lo] = acc_lo
                out_v[j * CHUNK_NODES + c, sl_hi] = acc_hi
                return carry

            lax.fori_loop(0, CHUNK_NODES, node_body, 0)

    for b in range(NBUF - 1):
        pltpu.async_copy(pk_hbm.at[idx_v.at[b]], rows_v.at[b], sems[b])

    def ring_body(t, carry):
        for b in range(NBUF):
            j = NBUF * t + b
            nxt = (b + NBUF - 1) % NBUF

            @pl.when(j + NBUF - 1 < nchunks)
            def _():
                pltpu.async_copy(pk_hbm.at[idx_v.at[j + NBUF - 1]],
                                 rows_v.at[nxt], sems[nxt])

            pltpu.make_async_copy(pk_hbm.at[idx_v.at[j]],
                                  rows_v.at[b], sems[b]).wait()
            compute(j, rows_v.at[b])
        return carry

    lax.fori_loop(0, nchunks // NBUF, ring_body, 0)

    @pl.when(cid == 0)
    def _():
        pltpu.sync_copy(out_v.at[pl.ds(0, CH0 * CHUNK_NODES)],
                        out_hbm.at[pl.ds(base * CHUNK_NODES, CH0 * CHUNK_NODES)])

    @pl.when(cid == 1)
    def _():
        pltpu.sync_copy(out_v.at[pl.ds(0, CH1 * CHUNK_NODES)],
                        out_hbm.at[pl.ds(base * CHUNK_NODES, CH1 * CHUNK_NODES)])


_BN = 1000


def _pack_bf16(x):
    y = lax.bitcast_convert_type(x, jnp.int32)
    r = y + jnp.int32(0x7FFF) + ((y >> 16) & 1)
    a = r[:, :HWORDS]
    b = r[:, HWORDS:]
    return ((a >> 16) & jnp.int32(0xFFFF)) | (b & jnp.int32(-65536))


def _inproj_body(x_ref, w_ref, b_ref, o_ref, pk_ref):
    acc = jnp.dot(x_ref[...], w_ref[...], preferred_element_type=jnp.float32)
    o = jnp.maximum(acc + b_ref[...], 0.0)
    o_ref[...] = o
    pk_ref[...] = _pack_bf16(o)


def _input_projection(x, w_t, b):
    return pl.pallas_call(
        _inproj_body,
        grid=(N_NODES // _BN,),
        in_specs=[
            pl.BlockSpec((_BN, IN_DIM), lambda i: (i, 0)),
            pl.BlockSpec((IN_DIM, HIDDEN), lambda i: (0, 0)),
            pl.BlockSpec((1, HIDDEN), lambda i: (0, 0)),
        ],
        out_specs=[
            pl.BlockSpec((_BN, HIDDEN), lambda i: (i, 0)),
            pl.BlockSpec((_BN, HWORDS), lambda i: (i, 0)),
        ],
        out_shape=[
            jax.ShapeDtypeStruct((N_NODES, HIDDEN), jnp.float32),
            jax.ShapeDtypeStruct((N_NODES, HWORDS), jnp.int32),
        ],
    )(x, w_t, b)


def _layerstep_body(agg_ref, w_ref, pre_ref, inf_ref, o_ref, pk_ref):
    fout = jnp.maximum(
        jnp.dot(agg_ref[...], w_ref[...], preferred_element_type=jnp.float32),
        0.0)
    o = ((1.0 - ALPHA - BETA) * fout
         + BETA * pre_ref[...] + ALPHA * inf_ref[...])
    o_ref[...] = o
    pk_ref[...] = _pack_bf16(o)


def _layer_step(agg, w_t, pre, inf):
    return pl.pallas_call(
        _layerstep_body,
        grid=(N_NODES // _BN,),
        in_specs=[
            pl.BlockSpec((_BN, HIDDEN), lambda i: (i, 0)),
            pl.BlockSpec((HIDDEN, HIDDEN), lambda i: (0, 0)),
            pl.BlockSpec((_BN, HIDDEN), lambda i: (i, 0)),
            pl.BlockSpec((_BN, HIDDEN), lambda i: (i, 0)),
        ],
        out_specs=[
            pl.BlockSpec((_BN, HIDDEN), lambda i: (i, 0)),
            pl.BlockSpec((_BN, HWORDS), lambda i: (i, 0)),
        ],
        out_shape=[
            jax.ShapeDtypeStruct((N_NODES, HIDDEN), jnp.float32),
            jax.ShapeDtypeStruct((N_NODES, HWORDS), jnp.int32),
        ],
    )(agg, w_t, pre, inf)


def _outproj_body(x_ref, w_ref, b_ref, o_ref):
    acc = jnp.dot(x_ref[...], w_ref[...], preferred_element_type=jnp.float32)
    o_ref[...] = jnp.maximum(acc + b_ref[...], 0.0)


def _output_projection(x, w_t, b):
    return pl.pallas_call(
        _outproj_body,
        grid=(N_NODES // _BN,),
        in_specs=[
            pl.BlockSpec((_BN, HIDDEN), lambda i: (i, 0)),
            pl.BlockSpec((HIDDEN, HIDDEN), lambda i: (0, 0)),
            pl.BlockSpec((1, HIDDEN), lambda i: (0, 0)),
        ],
        out_specs=pl.BlockSpec((_BN, HIDDEN), lambda i: (i, 0)),
        out_shape=jax.ShapeDtypeStruct((N_NODES, HIDDEN), jnp.float32),
    )(x, w_t, b)


def kernel(input_x, paths, path_types, W_in, b_in, layer_fc, path_w, W_out, b_out):
    n = input_x.shape[0]

    idx = jnp.transpose(paths, (1, 0, 2)).reshape(n, FAN)
    idx = jnp.pad(idx, ((0, N_PAD - n), (0, 0)))
    idx2d = idx.reshape(TOT_CHUNKS, ROWS_PER_CHUNK)

    mask = (path_types == 0).astype(jnp.float32)
    scale = mask / jnp.sum(mask)
    coefs = (scale[None, :, None, None] * path_w[:, 0][:, None, :, :]
             ).reshape(NUM_LAYERS, FAN, HIDDEN)

    in_feats, pk = _input_projection(input_x, W_in.T, b_in.reshape(1, HIDDEN))
    feats = in_feats
    for i in range(NUM_LAYERS):
        agg = _gather_reduce(pk, idx2d, coefs[i])
        feats, pk = _layer_step(agg, layer_fc[i].T, feats, in_feats)
    return _output_projection(feats, W_out.T, b_out.reshape(1, HIDDEN))


# speedup vs baseline: 1.0759x; 1.0759x over previous
import functools

import jax
import jax.numpy as jnp
from jax import lax
from jax.experimental import pallas as pl
from jax.experimental.pallas import tpu as pltpu
from jax.experimental.pallas import tpu_sc as plsc

N_NODES = 10000
IN_DIM = 256
HIDDEN = 128
HWORDS = HIDDEN // 2
NUM_LAYERS = 2
NUM_PATHS = 4
PATH_LEN = 4
ALPHA = 0.1
BETA = 0.2

_INFO = plsc.get_sparse_core_info()
_NC = _INFO.num_cores
_NS = _INFO.num_subcores
_NW = _NC * _NS

N_PAD = 10240
FAN = NUM_PATHS * PATH_LEN
CHUNK_NODES = 8
ROWS_PER_CHUNK = CHUNK_NODES * FAN
TOT_CHUNKS = N_PAD // CHUNK_NODES
IDX_W = CHUNK_NODES * PATH_LEN
NBUF = 4
CH0 = 60
CH1 = 20
CHMAX = max(CH0, CH1)


@functools.partial(
    pl.kernel,
    mesh=plsc.VectorSubcoreMesh(core_axis_name="c", subcore_axis_name="s"),
    compiler_params=pltpu.CompilerParams(needs_layout_passes=False,
                                         use_tc_tiling_on_sc=False),
    out_type=jax.ShapeDtypeStruct((N_PAD, HIDDEN), jnp.float32),
    scratch_types=[
        pltpu.VMEM((CHMAX, ROWS_PER_CHUNK), jnp.int32),
        pltpu.VMEM((FAN, HIDDEN), jnp.float32),
        pltpu.VMEM((NBUF, ROWS_PER_CHUNK, HWORDS), jnp.int32),
        pltpu.VMEM((CHMAX * CHUNK_NODES, HIDDEN), jnp.float32),
    ] + [pltpu.SemaphoreType.DMA] * NBUF,
)
def _gather_reduce(pk_hbm, idx_hbm, coef_hbm, out_hbm,
                   idx_v, coef_v, rows_v, out_v, *sems):
    sid = lax.axis_index("s")
    cid = lax.axis_index("c")
    base = sid * (CH0 + CH1) + cid * CH0
    nchunks = jnp.where(cid == 0, CH0, CH1)

    @pl.when(cid == 0)
    def _():
        pltpu.sync_copy(idx_hbm.at[pl.ds(base, CH0)], idx_v.at[pl.ds(0, CH0)])

    @pl.when(cid == 1)
    def _():
        pltpu.sync_copy(idx_hbm.at[pl.ds(base, CH1)], idx_v.at[pl.ds(0, CH1)])

    pltpu.sync_copy(coef_hbm, coef_v)

    def unpack2(w):
        return plsc.unpack(plsc.bitcast(w, jnp.bfloat16),
                           format=plsc.PackFormat.INTERLEAVED)

    def compute(j, rows):
        for v in range(HWORDS // 16):
            sw = pl.ds(v * 16, 16)
            sl_lo = pl.ds(v * 16, 16)
            sl_hi = pl.ds(HWORDS + v * 16, 16)
            clo = [coef_v[r, sl_lo] for r in range(FAN)]
            chi = [coef_v[r, sl_hi] for r in range(FAN)]

            def node_body(c, carry):
                lo, hi = unpack2(rows[c * FAN, sw])
                acc_lo = clo[0] * lo
                acc_hi = chi[0] * hi
                for r in range(1, FAN):
                    lo, hi = unpack2(rows[c * FAN + r, sw])
                    acc_lo = acc_lo + clo[r] * lo
                    acc_hi = acc_hi + chi[r] * hi
                out_v[j * CHUNK_NODES + c, sl_lo] = acc_lo
                out_v[j * CHUNK_NODES + c, sl_hi] = acc_hi
                return carry

            lax.fori_loop(0, CHUNK_NODES, node_body, 0)

    for b in range(NBUF - 1):
        pltpu.async_copy(pk_hbm.at[idx_v.at[b]], rows_v.at[b], sems[b])

    def ring_body(t, carry):
        for b in range(NBUF):
            j = NBUF * t + b
            nxt = (b + NBUF - 1) % NBUF

            @pl.when(j + NBUF - 1 < nchunks)
            def _():
                pltpu.async_copy(pk_hbm.at[idx_v.at[j + NBUF - 1]],
                                 rows_v.at[nxt], sems[nxt])

            pltpu.make_async_copy(pk_hbm.at[idx_v.at[j]],
                                  rows_v.at[b], sems[b]).wait()
            compute(j, rows_v.at[b])
        return carry

    lax.fori_loop(0, nchunks // NBUF, ring_body, 0)

    @pl.when(cid == 0)
    def _():
        pltpu.sync_copy(out_v.at[pl.ds(0, CH0 * CHUNK_NODES)],
                        out_hbm.at[pl.ds(base * CHUNK_NODES, CH0 * CHUNK_NODES)])

    @pl.when(cid == 1)
    def _():
        pltpu.sync_copy(out_v.at[pl.ds(0, CH1 * CHUNK_NODES)],
                        out_hbm.at[pl.ds(base * CHUNK_NODES, CH1 * CHUNK_NODES)])


_BN = 1000


def _pack_bf16(x):
    y = lax.bitcast_convert_type(x, jnp.int32)
    r = y + jnp.int32(0x7FFF) + ((y >> 16) & 1)
    a = r[:, :HWORDS]
    b = r[:, HWORDS:]
    return ((a >> 16) & jnp.int32(0xFFFF)) | (b & jnp.int32(-65536))


def _inproj_body(x_ref, w_ref, b_ref, o_ref, pk_ref):
    acc = jnp.dot(x_ref[...], w_ref[...], preferred_element_type=jnp.float32)
    o = jnp.maximum(acc + b_ref[...], 0.0)
    o_ref[...] = o
    pk_ref[...] = _pack_bf16(o)


def _input_projection(x, w_t, b):
    return pl.pallas_call(
        _inproj_body,
        grid=(N_NODES // _BN,),
        in_specs=[
            pl.BlockSpec((_BN, IN_DIM), lambda i: (i, 0)),
            pl.BlockSpec((IN_DIM, HIDDEN), lambda i: (0, 0)),
            pl.BlockSpec((1, HIDDEN), lambda i: (0, 0)),
        ],
        out_specs=[
            pl.BlockSpec((_BN, HIDDEN), lambda i: (i, 0)),
            pl.BlockSpec((_BN, HWORDS), lambda i: (i, 0)),
        ],
        out_shape=[
            jax.ShapeDtypeStruct((N_NODES, HIDDEN), jnp.float32),
            jax.ShapeDtypeStruct((N_NODES, HWORDS), jnp.int32),
        ],
    )(x, w_t, b)


def _layerstep_body(agg_ref, w_ref, pre_ref, inf_ref, o_ref, pk_ref):
    fout = jnp.maximum(
        jnp.dot(agg_ref[...], w_ref[...], preferred_element_type=jnp.float32),
        0.0)
    o = ((1.0 - ALPHA - BETA) * fout
         + BETA * pre_ref[...] + ALPHA * inf_ref[...])
    o_ref[...] = o
    pk_ref[...] = _pack_bf16(o)


def _layer_step(agg, w_t, pre, inf):
    return pl.pallas_call(
        _layerstep_body,
        grid=(N_NODES // _BN,),
        in_specs=[
            pl.BlockSpec((_BN, HIDDEN), lambda i: (i, 0)),
            pl.BlockSpec((HIDDEN, HIDDEN), lambda i: (0, 0)),
            pl.BlockSpec((_BN, HIDDEN), lambda i: (i, 0)),
            pl.BlockSpec((_BN, HIDDEN), lambda i: (i, 0)),
        ],
        out_specs=[
            pl.BlockSpec((_BN, HIDDEN), lambda i: (i, 0)),
            pl.BlockSpec((_BN, HWORDS), lambda i: (i, 0)),
        ],
        out_shape=[
            jax.ShapeDtypeStruct((N_NODES, HIDDEN), jnp.float32),
            jax.ShapeDtypeStruct((N_NODES, HWORDS), jnp.int32),
        ],
    )(agg, w_t, pre, inf)


def _final_body(agg_ref, w_ref, pre_ref, inf_ref, wo_ref, bo_ref, o_ref):
    fout = jnp.maximum(
        jnp.dot(agg_ref[...], w_ref[...], preferred_element_type=jnp.float32),
        0.0)
    feats = ((1.0 - ALPHA - BETA) * fout
             + BETA * pre_ref[...] + ALPHA * inf_ref[...])
    acc = jnp.dot(feats, wo_ref[...], preferred_element_type=jnp.float32)
    o_ref[...] = jnp.maximum(acc + bo_ref[...], 0.0)


def _final_step(agg, w_t, pre, inf, wo_t, bo):
    return pl.pallas_call(
        _final_body,
        grid=(N_NODES // _BN,),
        in_specs=[
            pl.BlockSpec((_BN, HIDDEN), lambda i: (i, 0)),
            pl.BlockSpec((HIDDEN, HIDDEN), lambda i: (0, 0)),
            pl.BlockSpec((_BN, HIDDEN), lambda i: (i, 0)),
            pl.BlockSpec((_BN, HIDDEN), lambda i: (i, 0)),
            pl.BlockSpec((HIDDEN, HIDDEN), lambda i: (0, 0)),
            pl.BlockSpec((1, HIDDEN), lambda i: (0, 0)),
        ],
        out_specs=pl.BlockSpec((_BN, HIDDEN), lambda i: (i, 0)),
        out_shape=jax.ShapeDtypeStruct((N_NODES, HIDDEN), jnp.float32),
    )(agg, w_t, pre, inf, wo_t, bo)


def kernel(input_x, paths, path_types, W_in, b_in, layer_fc, path_w, W_out, b_out):
    n = input_x.shape[0]

    idx = jnp.transpose(paths, (1, 0, 2)).reshape(n, FAN)
    idx = jnp.pad(idx, ((0, N_PAD - n), (0, 0)))
    idx2d = idx.reshape(TOT_CHUNKS, ROWS_PER_CHUNK)

    mask = (path_types == 0).astype(jnp.float32)
    scale = mask / jnp.sum(mask)
    coefs = (scale[None, :, None, None] * path_w[:, 0][:, None, :, :]
             ).reshape(NUM_LAYERS, FAN, HIDDEN)

    in_feats, pk = _input_projection(input_x, W_in.T, b_in.reshape(1, HIDDEN))
    feats = in_feats
    for i in range(NUM_LAYERS - 1):
        agg = _gather_reduce(pk, idx2d, coefs[i])
        feats, pk = _layer_step(agg, layer_fc[i].T, feats, in_feats)
    agg = _gather_reduce(pk, idx2d, coefs[NUM_LAYERS - 1])
    return _final_step(agg, layer_fc[NUM_LAYERS - 1].T, feats, in_feats,
                       W_out.T, b_out.reshape(1, HIDDEN))
